# baseline (speedup 1.0000x reference)
import jax
import jax.numpy as jnp
from jax.experimental import pallas as pl
from jax.experimental.pallas import tpu as pltpu

_D = 64
_O = 512
_LR = 0.01
_SIGMA = _O / 2.0
_INV2S2 = 1.0 / (2.0 * _SIGMA * _SIGMA)

_NC = 2
_OH = _O // _NC
_BI = 128
_NB = _OH // _BI
_BI2 = 64
_NB2 = _OH // _BI2

_HIGH = jax.lax.Precision.HIGHEST


def _stats_kernel(xb_ref, wT_ref, xTh_ref, winner_ref, b_ref, c_ref,
                  b_acc, nb_half):
    g = pl.program_id(1)
    w = wT_ref[...]
    xb = xb_ref[...]

    diff = w - xb[:, :, None]
    dist = jnp.sum(diff * diff, axis=1)
    nb = jnp.exp(dist * (-_INV2S2))

    minv = jnp.min(dist, axis=1, keepdims=True)
    iota = jax.lax.broadcasted_iota(jnp.int32, dist.shape, 1)
    win = jnp.min(jnp.where(dist == minv, iota, _O), axis=1, keepdims=True)
    winner_ref[...] = win

    nb_half[pl.ds(g * _BI, _BI), :] = nb

    b_part = jnp.sum(nb[:, None, :] * w, axis=0)

    @pl.when(g == 0)
    def _init():
        b_acc[...] = b_part

    @pl.when(g > 0)
    def _accum():
        b_acc[...] = b_acc[...] + b_part

    @pl.when(g == _NB - 1)
    def _emit():
        c_ref[0] = jax.lax.dot_general(
            xTh_ref[...], nb_half[...],
            dimension_numbers=(((1,), (0,)), ((), ())),
            preferred_element_type=jnp.float32,
            precision=_HIGH,
        )
        b_ref[0] = b_acc[...]


def _apply_kernel(wT_ref, b_ref, c_ref, out_ref, u_acc):
    g = pl.program_id(1)

    @pl.when(g == 0)
    def _make_u():
        bc = c_ref[...] - b_ref[...]
        u_acc[...] = _LR * (bc[0] + bc[1])

    out_ref[...] = wT_ref[...] + u_acc[...][None]


def kernel(x, weights):
    x = x.reshape(_O, _D)
    wT = weights.transpose(0, 2, 1)
    xT = x.T

    winner2d, b_parts, c_parts = pl.pallas_call(
        _stats_kernel,
        grid=(_NC, _NB),
        in_specs=[
            pl.BlockSpec((_BI, _D), lambda c, g: (c * _NB + g, 0)),
            pl.BlockSpec((_BI, _D, _O), lambda c, g: (c * _NB + g, 0, 0)),
            pl.BlockSpec((_D, _OH), lambda c, g: (0, c)),
        ],
        out_specs=[
            pl.BlockSpec((_BI, 1), lambda c, g: (c * _NB + g, 0)),
            pl.BlockSpec((1, _D, _O), lambda c, g: (c, 0, 0)),
            pl.BlockSpec((1, _D, _O), lambda c, g: (c, 0, 0)),
        ],
        out_shape=[
            jax.ShapeDtypeStruct((_O, 1), jnp.int32),
            jax.ShapeDtypeStruct((_NC, _D, _O), jnp.float32),
            jax.ShapeDtypeStruct((_NC, _D, _O), jnp.float32),
        ],
        scratch_shapes=[
            pltpu.VMEM((_D, _O), jnp.float32),
            pltpu.VMEM((_OH, _O), jnp.float32),
        ],
        compiler_params=pltpu.CompilerParams(
            dimension_semantics=("parallel", "arbitrary"),
        ),
    )(x, wT, xT)

    new_wT = pl.pallas_call(
        _apply_kernel,
        grid=(_NC, _NB2),
        in_specs=[
            pl.BlockSpec((_BI2, _D, _O), lambda c, g: (c * _NB2 + g, 0, 0)),
            pl.BlockSpec((_NC, _D, _O), lambda c, g: (0, 0, 0)),
            pl.BlockSpec((_NC, _D, _O), lambda c, g: (0, 0, 0)),
        ],
        out_specs=pl.BlockSpec((_BI2, _D, _O), lambda c, g: (c * _NB2 + g, 0, 0)),
        out_shape=jax.ShapeDtypeStruct((_O, _D, _O), jnp.float32),
        scratch_shapes=[pltpu.VMEM((_D, _O), jnp.float32)],
        compiler_params=pltpu.CompilerParams(
            dimension_semantics=("parallel", "arbitrary"),
        ),
    )(wT, b_parts, c_parts)

    return winner2d.reshape(_O), new_wT.transpose(0, 2, 1)


# speedup vs baseline: 1.0813x; 1.0813x over previous
import jax
import jax.numpy as jnp
from jax.experimental import pallas as pl
from jax.experimental.pallas import tpu as pltpu

_D = 64
_O = 512
_LR = 0.01
_SIGMA = _O / 2.0
_INV2S2 = 1.0 / (2.0 * _SIGMA * _SIGMA)

_BI = 64
_NP = _O // _BI
_CACHE_BLKS = 2

_HIGH = jax.lax.Precision.HIGHEST


def _fused_kernel(xb_ref, wT_ref, xT_ref, winner_ref, out_ref,
                  b_acc, nb_all, u_scr, w_cache, copy_sem):
    t = pl.program_id(0)

    @pl.when(t < _NP)
    def _phase1():
        g = t

        @pl.when(g < _CACHE_BLKS)
        def _cache_start():
            pltpu.make_async_copy(
                wT_ref, w_cache.at[pl.ds(g * _BI, _BI)], copy_sem
            ).start()

        w = wT_ref[...]
        xb = xb_ref[...]

        diff = w - xb[:, :, None]
        dist = jnp.sum(diff * diff, axis=1)
        nb = jnp.exp(dist * (-_INV2S2))

        minv = jnp.min(dist, axis=1, keepdims=True)
        iota = jax.lax.broadcasted_iota(jnp.int32, dist.shape, 1)
        win = jnp.min(jnp.where(dist == minv, iota, _O), axis=1, keepdims=True)
        winner_ref[...] = win

        nb_all[pl.ds(g * _BI, _BI), :] = nb

        b_part = jnp.sum(nb[:, None, :] * w, axis=0)

        @pl.when(g == 0)
        def _init():
            b_acc[...] = b_part

        @pl.when(g > 0)
        def _accum():
            b_acc[...] = b_acc[...] + b_part

        @pl.when(g < _CACHE_BLKS)
        def _cache_wait():
            pltpu.make_async_copy(
                wT_ref, w_cache.at[pl.ds(g * _BI, _BI)], copy_sem
            ).wait()

    @pl.when(t == _NP)
    def _make_u():
        cT = jax.lax.dot_general(
            xT_ref[...], nb_all[...],
            dimension_numbers=(((1,), (0,)), ((), ())),
            preferred_element_type=jnp.float32,
            precision=_HIGH,
        )
        u_scr[...] = _LR * (cT - b_acc[...])

    @pl.when(t >= _NP)
    def _phase2():
        u = u_scr[...][None]

        @pl.when(jnp.logical_or(t == _NP, t > _NP + _CACHE_BLKS))
        def _from_stream():
            out_ref[...] = wT_ref[...] + u

        for k in range(_CACHE_BLKS):
            @pl.when(t == _NP + 1 + k)
            def _from_cache():
                out_ref[...] = w_cache[pl.ds(k * _BI, _BI)] + u


def _w_index(t):
    return (jnp.where(t < _NP, t,
                      jnp.where(t <= _NP + _CACHE_BLKS, _NP - 1, t - _NP - 1)),
            0, 0)


def _row_block(t):
    return jnp.where(t == _NP, _NP - 1, t - _NP - 1)


def kernel(x, weights):
    x = x.reshape(_O, _D)
    wT = weights.transpose(0, 2, 1)
    xT = x.T

    winner2d, new_wT = pl.pallas_call(
        _fused_kernel,
        grid=(2 * _NP,),
        in_specs=[
            pl.BlockSpec((_BI, _D), lambda t: (jnp.where(t < _NP, t, 0), 0)),
            pl.BlockSpec((_BI, _D, _O), _w_index),
            pl.BlockSpec((_D, _O), lambda t: (0, 0)),
        ],
        out_specs=[
            pl.BlockSpec((_BI, 1),
                         lambda t: (jnp.where(t < _NP, t, _NP - 1), 0)),
            pl.BlockSpec((_BI, _D, _O),
                         lambda t: (jnp.where(t < _NP, _NP - 1, _row_block(t)),
                                    0, 0)),
        ],
        out_shape=[
            jax.ShapeDtypeStruct((_O, 1), jnp.int32),
            jax.ShapeDtypeStruct((_O, _D, _O), jnp.float32),
        ],
        scratch_shapes=[
            pltpu.VMEM((_D, _O), jnp.float32),
            pltpu.VMEM((_O, _O), jnp.float32),
            pltpu.VMEM((_D, _O), jnp.float32),
            pltpu.VMEM((_CACHE_BLKS * _BI, _D, _O), jnp.float32),
            pltpu.SemaphoreType.DMA,
        ],
    )(x, wT, xT)

    return winner2d.reshape(_O), new_wT.transpose(0, 2, 1)


# speedup vs baseline: 1.0903x; 1.0084x over previous
import jax
import jax.numpy as jnp
from jax.experimental import pallas as pl
from jax.experimental.pallas import tpu as pltpu

_D = 64
_O = 512
_LR = 0.01
_SIGMA = _O / 2.0
_INV2S2 = 1.0 / (2.0 * _SIGMA * _SIGMA)

_BI = 32
_NP = _O // _BI
_CACHE_BLKS = 9

_HIGH = jax.lax.Precision.HIGHEST


def _fused_kernel(xb_ref, wT_ref, xT_ref, winner_ref, out_ref,
                  b_acc, nb_all, u_scr, w_cache, copy_sem):
    t = pl.program_id(0)

    @pl.when(t < _NP)
    def _phase1():
        g = t

        @pl.when(g < _CACHE_BLKS)
        def _cache_start():
            pltpu.make_async_copy(
                wT_ref, w_cache.at[pl.ds(g * _BI, _BI)], copy_sem
            ).start()

        w = wT_ref[...]
        xb = xb_ref[...]

        diff = w - xb[:, :, None]
        dist = jnp.sum(diff * diff, axis=1)
        nb = jnp.exp(dist * (-_INV2S2))

        minv = jnp.min(dist, axis=1, keepdims=True)
        iota = jax.lax.broadcasted_iota(jnp.int32, dist.shape, 1)
        win = jnp.min(jnp.where(dist == minv, iota, _O), axis=1, keepdims=True)
        winner_ref[...] = win

        nb_all[pl.ds(g * _BI, _BI), :] = nb

        b_part = jnp.sum(nb[:, None, :] * w, axis=0)

        @pl.when(g == 0)
        def _init():
            b_acc[...] = b_part

        @pl.when(g > 0)
        def _accum():
            b_acc[...] = b_acc[...] + b_part

        @pl.when(g < _CACHE_BLKS)
        def _cache_wait():
            pltpu.make_async_copy(
                wT_ref, w_cache.at[pl.ds(g * _BI, _BI)], copy_sem
            ).wait()

    @pl.when(t == _NP)
    def _make_u():
        cT = jax.lax.dot_general(
            xT_ref[...], nb_all[...],
            dimension_numbers=(((1,), (0,)), ((), ())),
            preferred_element_type=jnp.float32,
            precision=_HIGH,
        )
        u_scr[...] = _LR * (cT - b_acc[...])

    @pl.when(t >= _NP)
    def _phase2():
        u = u_scr[...][None]

        @pl.when(jnp.logical_or(t == _NP, t > _NP + _CACHE_BLKS))
        def _from_stream():
            out_ref[...] = wT_ref[...] + u

        for k in range(_CACHE_BLKS):
            @pl.when(t == _NP + 1 + k)
            def _from_cache():
                out_ref[...] = w_cache[pl.ds(k * _BI, _BI)] + u


def _w_index(t):
    return (jnp.where(t < _NP, t,
                      jnp.where(t <= _NP + _CACHE_BLKS, _NP - 1, t - _NP - 1)),
            0, 0)


def _row_block(t):
    return jnp.where(t == _NP, _NP - 1, t - _NP - 1)


def kernel(x, weights):
    x = x.reshape(_O, _D)
    wT = weights.transpose(0, 2, 1)
    xT = x.T

    winner2d, new_wT = pl.pallas_call(
        _fused_kernel,
        grid=(2 * _NP,),
        in_specs=[
            pl.BlockSpec((_BI, _D), lambda t: (jnp.where(t < _NP, t, 0), 0)),
            pl.BlockSpec((_BI, _D, _O), _w_index),
            pl.BlockSpec((_D, _O), lambda t: (0, 0)),
        ],
        out_specs=[
            pl.BlockSpec((_BI, 1),
                         lambda t: (jnp.where(t < _NP, t, _NP - 1), 0)),
            pl.BlockSpec((_BI, _D, _O),
                         lambda t: (jnp.where(t < _NP, _NP - 1, _row_block(t)),
                                    0, 0)),
        ],
        out_shape=[
            jax.ShapeDtypeStruct((_O, 1), jnp.int32),
            jax.ShapeDtypeStruct((_O, _D, _O), jnp.float32),
        ],
        scratch_shapes=[
            pltpu.VMEM((_D, _O), jnp.float32),
            pltpu.VMEM((_O, _O), jnp.float32),
            pltpu.VMEM((_D, _O), jnp.float32),
            pltpu.VMEM((_CACHE_BLKS * _BI, _D, _O), jnp.float32),
            pltpu.SemaphoreType.DMA,
        ],
    )(x, wT, xT)

    return winner2d.reshape(_O), new_wT.transpose(0, 2, 1)
